# baseline (speedup 1.0000x reference)
import functools

import jax
import jax.numpy as jnp
from jax.experimental import pallas as pl
from jax.experimental.pallas import tpu as pltpu
from jax.experimental.pallas import tpu_sc as plsc

_CONF_THRESH = 0.5
_IOU_THRESH = 0.3
_VAR0, _VAR1 = 0.1, 0.2
_MAX_DET = 100
_NTILES = 16
_PER = 1280
_VECS = _PER // 16


def _sc_body(n_valid,
             l0, l1, l2, l3, c1, p0, p1, p2, p3,
             out_hbm,
             s_ref, x1_ref, y1_ref, x2_ref, y2_ref, ar_ref,
             i0, i1, i2, i3, i4, i5, i6, i7, i8,
             tab_ref, row_ref, outl_ref,
             shtab_ref,
             ):
    t = jax.lax.axis_index("s")
    base = t * _PER

    srcs = (l0, l1, l2, l3, c1, p0, p1, p2, p3)
    dsts = (i0, i1, i2, i3, i4, i5, i6, i7, i8)
    for src, dst in zip(srcs, dsts):
        pltpu.sync_copy(src.at[pl.ds(base, _PER)], dst)

    lane = jax.lax.iota(jnp.int32, 16)

    def decode(i, _):
        sl = pl.ds(i * 16, 16)
        p2v = i7[sl]
        p3v = i8[sl]
        cx = i5[sl] + i0[sl] * _VAR0 * p2v
        cy = i6[sl] + i1[sl] * _VAR0 * p3v
        w = p2v * jnp.exp(i2[sl] * _VAR1)
        h = p3v * jnp.exp(i3[sl] * _VAR1)
        x1 = cx - w / 2.0
        y1 = cy - h / 2.0
        x2 = cx + w / 2.0
        y2 = cy + h / 2.0
        cv = i4[sl]
        prob = 1.0 / (1.0 + jnp.exp(-cv))
        score = jnp.where(prob >= _CONF_THRESH, prob, 0.0)
        gidx = base + i * 16 + lane
        score = jnp.where(gidx < n_valid, score, 0.0)
        x1_ref[sl] = x1
        y1_ref[sl] = y1
        x2_ref[sl] = x2
        y2_ref[sl] = y2
        ar_ref[sl] = jnp.maximum(x2 - x1, 0.0) * jnp.maximum(y2 - y1, 0.0)
        s_ref[sl] = score
        return 0

    jax.lax.fori_loop(0, _VECS, decode, 0, unroll=4)

    bigr = jnp.int32(_VECS + 1)

    def pick(i, wcarry):
        ws, wi, wx1, wy1, wx2, wy2, war = wcarry

        def sweep(j, carry):
            m, r = carry
            sl = pl.ds(j * 16, 16)
            s = s_ref[sl]
            x1 = x1_ref[sl]
            y1 = y1_ref[sl]
            x2 = x2_ref[sl]
            y2 = y2_ref[sl]
            ar = ar_ref[sl]
            xx1 = jnp.maximum(wx1, x1)
            yy1 = jnp.maximum(wy1, y1)
            xx2 = jnp.minimum(wx2, x2)
            yy2 = jnp.minimum(wy2, y2)
            iw = jnp.maximum(xx2 - xx1, 0.0)
            ih = jnp.maximum(yy2 - yy1, 0.0)
            inter = iw * ih
            iou = inter / (war + ar - inter + 1e-9)
            gidx = base + j * 16 + lane
            supp = jnp.logical_or(iou > _IOU_THRESH, gidx == wi)
            s = jnp.where(supp, -1.0, s)
            s_ref[sl] = s
            upd = s > m
            m = jnp.where(upd, s, m)
            r = jnp.where(upd, j, r)
            return m, r

        m0 = jnp.full((16,), -2.0, jnp.float32)
        r0 = jnp.full((16,), bigr, jnp.int32)
        m, r = jax.lax.fori_loop(0, _VECS, sweep, (m0, r0), unroll=8)

        def bfly(v, op):
            for sh in (8, 4, 2, 1):
                v = op(v, v.at[lane ^ sh].get(mode='promise_in_bounds'))
            return v

        mm = bfly(m, jnp.maximum)
        ls = mm[0]
        rmask = m == mm
        rv = bfly(jnp.where(rmask, r, bigr), jnp.minimum)
        rbest = rv[0]
        lmask = jnp.logical_and(rmask, r == rv)
        lv = bfly(jnp.where(lmask, lane, 127), jnp.minimum)
        lbest = lv[0]
        lidx = rbest * 16 + lbest
        zero = jnp.float32(0.0)
        rsl = pl.ds(rbest * 16, 16)
        gm = 'promise_in_bounds'
        bx1 = x1_ref[rsl].at[lv].get(mode=gm)[0]
        by1 = y1_ref[rsl].at[lv].get(mode=gm)[0]
        bx2 = x2_ref[rsl].at[lv].get(mode=gm)[0]
        by2 = y2_ref[rsl].at[lv].get(mode=gm)[0]
        bar = ar_ref[rsl].at[lv].get(mode=gm)[0]
        rowv = (jnp.where(lane == 0, ls, zero)
                + jnp.where(lane == 1, (base + lidx).astype(jnp.float32), zero)
                + jnp.where(lane == 2, bx1, zero)
                + jnp.where(lane == 3, by1, zero)
                + jnp.where(lane == 4, bx2, zero)
                + jnp.where(lane == 5, by2, zero)
                + jnp.where(lane == 6, bar, zero))
        row_ref[...] = rowv

        pltpu.sync_copy(row_ref, shtab_ref.at[pl.ds(t * 16, 16)])
        plsc.subcore_barrier()
        pltpu.sync_copy(shtab_ref, tab_ref)
        plsc.subcore_barrier()

        def red(u, best):
            bs, bi, bj = best
            tv = tab_ref[pl.ds(u * 16, 16)]
            cs = tv[0]
            ci = tv[1]
            better = jnp.logical_or(cs > bs,
                                    jnp.logical_and(cs == bs, ci < bi))
            return (jnp.where(better, cs, bs),
                    jnp.where(better, ci, bi),
                    jnp.where(better, u, bj))

        bs, bi, bj = jax.lax.fori_loop(
            0, _NTILES, red,
            (jnp.float32(-3.0), jnp.float32(3.0e7), jnp.int32(0)),
            unroll=_NTILES)

        tvb = tab_ref[pl.ds(bj * 16, 16)]
        nwx1 = tvb[2]
        nwy1 = tvb[3]
        nwx2 = tvb[4]
        nwy2 = tvb[5]
        nwar = tvb[6]
        valid = bs > 0.0

        @pl.when(t == 0)
        def _():
            rowv = (jnp.where(lane == 0, nwx1, zero)
                    + jnp.where(lane == 1, nwy1, zero)
                    + jnp.where(lane == 2, nwx2, zero)
                    + jnp.where(lane == 3, nwy2, zero)
                    + jnp.where(lane == 4, bs, zero))
            outl_ref[pl.ds(i * 16, 16)] = jnp.where(
                valid, rowv, jnp.full((16,), 0.0, jnp.float32))

        return (bs, bi.astype(jnp.int32), nwx1, nwy1, nwx2, nwy2, nwar)

    init = (jnp.float32(0.0), jnp.int32(-1),
            jnp.float32(0.0), jnp.float32(0.0),
            jnp.float32(0.0), jnp.float32(0.0), jnp.float32(0.0))
    jax.lax.fori_loop(0, _MAX_DET, pick, init)

    @pl.when(t == 0)
    def _():
        pltpu.sync_copy(outl_ref, out_hbm)


@jax.jit
def kernel(loc, conf, priors):
    n = loc.shape[0]
    n_pad = _NTILES * _PER

    def col(a, j, fill):
        c = a[:, j]
        return jnp.concatenate([c, jnp.full((n_pad - n,), fill, c.dtype)])

    args = (
        col(loc, 0, 0.0), col(loc, 1, 0.0), col(loc, 2, 0.0), col(loc, 3, 0.0),
        col(conf, 1, -1e9),
        col(priors, 0, 0.0), col(priors, 1, 0.0), col(priors, 2, 0.0), col(priors, 3, 0.0),
    )

    scratch = [
        pltpu.VMEM((_PER,), jnp.float32),
        pltpu.VMEM((_PER,), jnp.float32),
        pltpu.VMEM((_PER,), jnp.float32),
        pltpu.VMEM((_PER,), jnp.float32),
        pltpu.VMEM((_PER,), jnp.float32),
        pltpu.VMEM((_PER,), jnp.float32),
    ] + [pltpu.VMEM((_PER,), jnp.float32)] * 9 + [
        pltpu.VMEM((_NTILES * 16,), jnp.float32),
        pltpu.VMEM((16,), jnp.float32),
        pltpu.VMEM((_MAX_DET * 16,), jnp.float32),
        pltpu.VMEM_SHARED((_NTILES * 16,), jnp.float32),
    ]

    out = pl.kernel(
        functools.partial(_sc_body, n),
        out_type=jax.ShapeDtypeStruct((_MAX_DET * 16,), jnp.float32),
        mesh=plsc.VectorSubcoreMesh(
            core_axis_name="c", subcore_axis_name="s", num_cores=1),
        scratch_types=scratch,
    )(*args)
    return out.reshape(_MAX_DET, 16)[:, :5]


# speedup vs baseline: 2.1259x; 2.1259x over previous
import functools

import jax
import jax.numpy as jnp
from jax.experimental import pallas as pl
from jax.experimental.pallas import tpu as pltpu
from jax.experimental.pallas import tpu_sc as plsc

_CONF_THRESH = 0.5
_IOU_THRESH = 0.3
_VAR0, _VAR1 = 0.1, 0.2
_MAX_DET = 100
_NTILES = 16
_PER = 1280
_VECS = _PER // 16


def _sc_body(n_valid,
             l0, l1, l2, l3, c1, p0, p1, p2, p3,
             out_hbm,
             s_ref, x1_ref, y1_ref, x2_ref, y2_ref, ar_ref,
             i0, i1, i2, i3, i4, i5, i6, i7, i8,
             tab_ref, row_ref, outl_ref,
             shtab_ref,
             ):
    t = jax.lax.axis_index("s")
    base = t * _PER

    srcs = (l0, l1, l2, l3, c1, p0, p1, p2, p3)
    dsts = (i0, i1, i2, i3, i4, i5, i6, i7, i8)
    for src, dst in zip(srcs, dsts):
        pltpu.sync_copy(src.at[pl.ds(base, _PER)], dst)

    lane = jax.lax.iota(jnp.int32, 16)

    def decode(i, _):
        sl = pl.ds(i * 16, 16)
        p2v = i7[sl]
        p3v = i8[sl]
        cx = i5[sl] + i0[sl] * _VAR0 * p2v
        cy = i6[sl] + i1[sl] * _VAR0 * p3v
        w = p2v * jnp.exp(i2[sl] * _VAR1)
        h = p3v * jnp.exp(i3[sl] * _VAR1)
        x1 = cx - w / 2.0
        y1 = cy - h / 2.0
        x2 = cx + w / 2.0
        y2 = cy + h / 2.0
        cv = i4[sl]
        prob = 1.0 / (1.0 + jnp.exp(-cv))
        score = jnp.where(prob >= _CONF_THRESH, prob, 0.0)
        gidx = base + i * 16 + lane
        score = jnp.where(gidx < n_valid, score, 0.0)
        x1_ref[sl] = x1
        y1_ref[sl] = y1
        x2_ref[sl] = x2
        y2_ref[sl] = y2
        ar_ref[sl] = jnp.maximum(x2 - x1, 0.0) * jnp.maximum(y2 - y1, 0.0)
        s_ref[sl] = score
        return 0

    jax.lax.fori_loop(0, _VECS, decode, 0, unroll=4)

    bigr = jnp.int32(_VECS + 1)

    def pick(i, wcarry):
        ws, wi, wx1, wy1, wx2, wy2, war = wcarry

        def sweep(j, carry):
            m, r = carry
            sl = pl.ds(j * 16, 16)
            s = s_ref[sl]
            x1 = x1_ref[sl]
            y1 = y1_ref[sl]
            x2 = x2_ref[sl]
            y2 = y2_ref[sl]
            ar = ar_ref[sl]
            xx1 = jnp.maximum(wx1, x1)
            yy1 = jnp.maximum(wy1, y1)
            xx2 = jnp.minimum(wx2, x2)
            yy2 = jnp.minimum(wy2, y2)
            iw = jnp.maximum(xx2 - xx1, 0.0)
            ih = jnp.maximum(yy2 - yy1, 0.0)
            inter = iw * ih
            iou = inter / (war + ar - inter + 1e-9)
            gidx = base + j * 16 + lane
            supp = jnp.logical_or(iou > _IOU_THRESH, gidx == wi)
            s = jnp.where(supp, -1.0, s)
            s_ref[sl] = s
            upd = s > m
            m = jnp.where(upd, s, m)
            r = jnp.where(upd, j, r)
            return m, r

        m0 = jnp.full((16,), -2.0, jnp.float32)
        r0 = jnp.full((16,), bigr, jnp.int32)
        m, r = jax.lax.fori_loop(0, _VECS, sweep, (m0, r0), unroll=4)

        def bfly(v, op):
            for sh in (8, 4, 2, 1):
                v = op(v, v.at[lane ^ sh].get(mode='promise_in_bounds'))
            return v

        mm = bfly(m, jnp.maximum)
        ls = mm[0]
        rmask = m == mm
        rv = bfly(jnp.where(rmask, r, bigr), jnp.minimum)
        rbest = rv[0]
        lmask = jnp.logical_and(rmask, r == rv)
        lv = bfly(jnp.where(lmask, lane, 127), jnp.minimum)
        lbest = lv[0]
        lidx = rbest * 16 + lbest
        zero = jnp.float32(0.0)
        rsl = pl.ds(rbest * 16, 16)
        gm = 'promise_in_bounds'
        bx1 = x1_ref[rsl].at[lv].get(mode=gm)[0]
        by1 = y1_ref[rsl].at[lv].get(mode=gm)[0]
        bx2 = x2_ref[rsl].at[lv].get(mode=gm)[0]
        by2 = y2_ref[rsl].at[lv].get(mode=gm)[0]
        bar = ar_ref[rsl].at[lv].get(mode=gm)[0]
        rowv = (jnp.where(lane == 0, ls, zero)
                + jnp.where(lane == 1, (base + lidx).astype(jnp.float32), zero)
                + jnp.where(lane == 2, bx1, zero)
                + jnp.where(lane == 3, by1, zero)
                + jnp.where(lane == 4, bx2, zero)
                + jnp.where(lane == 5, by2, zero)
                + jnp.where(lane == 6, bar, zero))
        row_ref[...] = rowv

        pltpu.sync_copy(row_ref, shtab_ref.at[pl.ds(t * 16, 16)])
        plsc.subcore_barrier()
        pltpu.sync_copy(shtab_ref, tab_ref)
        plsc.subcore_barrier()

        def red(u, best):
            bs, bi, bj = best
            tv = tab_ref[pl.ds(u * 16, 16)]
            cs = tv[0]
            ci = tv[1]
            better = jnp.logical_or(cs > bs,
                                    jnp.logical_and(cs == bs, ci < bi))
            return (jnp.where(better, cs, bs),
                    jnp.where(better, ci, bi),
                    jnp.where(better, u, bj))

        bs, bi, bj = jax.lax.fori_loop(
            0, _NTILES, red,
            (jnp.float32(-3.0), jnp.float32(3.0e7), jnp.int32(0)))

        tvb = tab_ref[pl.ds(bj * 16, 16)]
        nwx1 = tvb[2]
        nwy1 = tvb[3]
        nwx2 = tvb[4]
        nwy2 = tvb[5]
        nwar = tvb[6]
        valid = bs > 0.0

        @pl.when(t == 0)
        def _():
            rowv = (jnp.where(lane == 0, nwx1, zero)
                    + jnp.where(lane == 1, nwy1, zero)
                    + jnp.where(lane == 2, nwx2, zero)
                    + jnp.where(lane == 3, nwy2, zero)
                    + jnp.where(lane == 4, bs, zero))
            outl_ref[pl.ds(i * 16, 16)] = jnp.where(
                valid, rowv, jnp.full((16,), 0.0, jnp.float32))

        return (bs, bi.astype(jnp.int32), nwx1, nwy1, nwx2, nwy2, nwar)

    init = (jnp.float32(0.0), jnp.int32(-1),
            jnp.float32(0.0), jnp.float32(0.0),
            jnp.float32(0.0), jnp.float32(0.0), jnp.float32(0.0))
    jax.lax.fori_loop(0, _MAX_DET, pick, init)

    @pl.when(t == 0)
    def _():
        pltpu.sync_copy(outl_ref, out_hbm)


@jax.jit
def kernel(loc, conf, priors):
    n = loc.shape[0]
    n_pad = _NTILES * _PER

    def col(a, j, fill):
        c = a[:, j]
        return jnp.concatenate([c, jnp.full((n_pad - n,), fill, c.dtype)])

    args = (
        col(loc, 0, 0.0), col(loc, 1, 0.0), col(loc, 2, 0.0), col(loc, 3, 0.0),
        col(conf, 1, -1e9),
        col(priors, 0, 0.0), col(priors, 1, 0.0), col(priors, 2, 0.0), col(priors, 3, 0.0),
    )

    scratch = [
        pltpu.VMEM((_PER,), jnp.float32),
        pltpu.VMEM((_PER,), jnp.float32),
        pltpu.VMEM((_PER,), jnp.float32),
        pltpu.VMEM((_PER,), jnp.float32),
        pltpu.VMEM((_PER,), jnp.float32),
        pltpu.VMEM((_PER,), jnp.float32),
    ] + [pltpu.VMEM((_PER,), jnp.float32)] * 9 + [
        pltpu.VMEM((_NTILES * 16,), jnp.float32),
        pltpu.VMEM((16,), jnp.float32),
        pltpu.VMEM((_MAX_DET * 16,), jnp.float32),
        pltpu.VMEM_SHARED((_NTILES * 16,), jnp.float32),
    ]

    out = pl.kernel(
        functools.partial(_sc_body, n),
        out_type=jax.ShapeDtypeStruct((_MAX_DET * 16,), jnp.float32),
        mesh=plsc.VectorSubcoreMesh(
            core_axis_name="c", subcore_axis_name="s", num_cores=1),
        scratch_types=scratch,
    )(*args)
    return out.reshape(_MAX_DET, 16)[:, :5]


# speedup vs baseline: 2.1934x; 1.0318x over previous
import functools

import jax
import jax.numpy as jnp
from jax.experimental import pallas as pl
from jax.experimental.pallas import tpu as pltpu
from jax.experimental.pallas import tpu_sc as plsc

_CONF_THRESH = 0.5
_IOU_THRESH = 0.3
_VAR0, _VAR1 = 0.1, 0.2
_MAX_DET = 100
_NTILES = 16
_PER = 1280
_VECS = _PER // 16


def _sc_body(n_valid,
             l0, l1, l2, l3, c1, p0, p1, p2, p3,
             out_hbm,
             s_ref, x1_ref, y1_ref, x2_ref, y2_ref, ar_ref,
             i0, i1, i2, i3, i4, i5, i6, i7, i8,
             tab_ref, row_ref, outl_ref,
             shtab_ref,
             ):
    t = jax.lax.axis_index("s")
    base = t * _PER

    srcs = (l0, l1, l2, l3, c1, p0, p1, p2, p3)
    dsts = (i0, i1, i2, i3, i4, i5, i6, i7, i8)
    for src, dst in zip(srcs, dsts):
        pltpu.sync_copy(src.at[pl.ds(base, _PER)], dst)

    lane = jax.lax.iota(jnp.int32, 16)

    def decode(i, _):
        sl = pl.ds(i * 16, 16)
        p2v = i7[sl]
        p3v = i8[sl]
        cx = i5[sl] + i0[sl] * _VAR0 * p2v
        cy = i6[sl] + i1[sl] * _VAR0 * p3v
        w = p2v * jnp.exp(i2[sl] * _VAR1)
        h = p3v * jnp.exp(i3[sl] * _VAR1)
        x1 = cx - w / 2.0
        y1 = cy - h / 2.0
        x2 = cx + w / 2.0
        y2 = cy + h / 2.0
        cv = i4[sl]
        prob = 1.0 / (1.0 + jnp.exp(-cv))
        score = jnp.where(prob >= _CONF_THRESH, prob, 0.0)
        gidx = base + i * 16 + lane
        score = jnp.where(gidx < n_valid, score, 0.0)
        x1_ref[sl] = x1
        y1_ref[sl] = y1
        x2_ref[sl] = x2
        y2_ref[sl] = y2
        ar_ref[sl] = jnp.maximum(x2 - x1, 0.0) * jnp.maximum(y2 - y1, 0.0)
        s_ref[sl] = score
        return 0

    jax.lax.fori_loop(0, _VECS, decode, 0, unroll=4)

    bigr = jnp.int32(_VECS + 1)

    def pick(i, wcarry):
        ws, wi, wx1, wy1, wx2, wy2, war = wcarry

        def sweep(j, carry):
            m, r = carry
            sl = pl.ds(j * 16, 16)
            s = s_ref[sl]
            x1 = x1_ref[sl]
            y1 = y1_ref[sl]
            x2 = x2_ref[sl]
            y2 = y2_ref[sl]
            ar = ar_ref[sl]
            xx1 = jnp.maximum(wx1, x1)
            yy1 = jnp.maximum(wy1, y1)
            xx2 = jnp.minimum(wx2, x2)
            yy2 = jnp.minimum(wy2, y2)
            iw = jnp.maximum(xx2 - xx1, 0.0)
            ih = jnp.maximum(yy2 - yy1, 0.0)
            inter = iw * ih
            iou = inter / (war + ar - inter + 1e-9)
            gidx = base + j * 16 + lane
            supp = jnp.logical_or(iou > _IOU_THRESH, gidx == wi)
            s = jnp.where(supp, -1.0, s)
            s_ref[sl] = s
            upd = s > m
            m = jnp.where(upd, s, m)
            r = jnp.where(upd, j, r)
            return m, r

        m0 = jnp.full((16,), -2.0, jnp.float32)
        r0 = jnp.full((16,), bigr, jnp.int32)
        m, r = jax.lax.fori_loop(0, _VECS, sweep, (m0, r0), unroll=4)

        def bfly(v, op):
            for sh in (8, 4, 2, 1):
                v = op(v, v.at[lane ^ sh].get(mode='promise_in_bounds'))
            return v

        mm = bfly(m, jnp.maximum)
        ls = mm[0]
        rmask = m == mm
        rv = bfly(jnp.where(rmask, r, bigr), jnp.minimum)
        rbest = rv[0]
        lmask = jnp.logical_and(rmask, r == rv)
        lv = bfly(jnp.where(lmask, lane, 127), jnp.minimum)
        lbest = lv[0]
        lidx = rbest * 16 + lbest
        zero = jnp.float32(0.0)
        rsl = pl.ds(rbest * 16, 16)
        gm = 'promise_in_bounds'
        bx1 = x1_ref[rsl].at[lv].get(mode=gm)[0]
        by1 = y1_ref[rsl].at[lv].get(mode=gm)[0]
        bx2 = x2_ref[rsl].at[lv].get(mode=gm)[0]
        by2 = y2_ref[rsl].at[lv].get(mode=gm)[0]
        bar = ar_ref[rsl].at[lv].get(mode=gm)[0]
        rowv = (jnp.where(lane == 0, ls, zero)
                + jnp.where(lane == 1, (base + lidx).astype(jnp.float32), zero)
                + jnp.where(lane == 2, bx1, zero)
                + jnp.where(lane == 3, by1, zero)
                + jnp.where(lane == 4, bx2, zero)
                + jnp.where(lane == 5, by2, zero)
                + jnp.where(lane == 6, bar, zero))
        row_ref[...] = rowv

        off = (i % 2) * (_NTILES * 16)
        pltpu.sync_copy(row_ref, shtab_ref.at[pl.ds(off + t * 16, 16)])
        plsc.subcore_barrier()
        pltpu.sync_copy(shtab_ref.at[pl.ds(off, _NTILES * 16)], tab_ref)

        def red(u, best):
            bs, bi, bj = best
            tv = tab_ref[pl.ds(u * 16, 16)]
            cs = tv[0]
            ci = tv[1]
            better = jnp.logical_or(cs > bs,
                                    jnp.logical_and(cs == bs, ci < bi))
            return (jnp.where(better, cs, bs),
                    jnp.where(better, ci, bi),
                    jnp.where(better, u, bj))

        bs, bi, bj = jax.lax.fori_loop(
            0, _NTILES, red,
            (jnp.float32(-3.0), jnp.float32(3.0e7), jnp.int32(0)))

        tvb = tab_ref[pl.ds(bj * 16, 16)]
        nwx1 = tvb[2]
        nwy1 = tvb[3]
        nwx2 = tvb[4]
        nwy2 = tvb[5]
        nwar = tvb[6]
        valid = bs > 0.0

        @pl.when(t == 0)
        def _():
            rowv = (jnp.where(lane == 0, nwx1, zero)
                    + jnp.where(lane == 1, nwy1, zero)
                    + jnp.where(lane == 2, nwx2, zero)
                    + jnp.where(lane == 3, nwy2, zero)
                    + jnp.where(lane == 4, bs, zero))
            outl_ref[pl.ds(i * 16, 16)] = jnp.where(
                valid, rowv, jnp.full((16,), 0.0, jnp.float32))

        return (bs, bi.astype(jnp.int32), nwx1, nwy1, nwx2, nwy2, nwar)

    init = (jnp.float32(0.0), jnp.int32(-1),
            jnp.float32(0.0), jnp.float32(0.0),
            jnp.float32(0.0), jnp.float32(0.0), jnp.float32(0.0))
    jax.lax.fori_loop(0, _MAX_DET, pick, init)

    @pl.when(t == 0)
    def _():
        pltpu.sync_copy(outl_ref, out_hbm)


@jax.jit
def kernel(loc, conf, priors):
    n = loc.shape[0]
    n_pad = _NTILES * _PER

    def col(a, j, fill):
        c = a[:, j]
        return jnp.concatenate([c, jnp.full((n_pad - n,), fill, c.dtype)])

    args = (
        col(loc, 0, 0.0), col(loc, 1, 0.0), col(loc, 2, 0.0), col(loc, 3, 0.0),
        col(conf, 1, -1e9),
        col(priors, 0, 0.0), col(priors, 1, 0.0), col(priors, 2, 0.0), col(priors, 3, 0.0),
    )

    scratch = [
        pltpu.VMEM((_PER,), jnp.float32),
        pltpu.VMEM((_PER,), jnp.float32),
        pltpu.VMEM((_PER,), jnp.float32),
        pltpu.VMEM((_PER,), jnp.float32),
        pltpu.VMEM((_PER,), jnp.float32),
        pltpu.VMEM((_PER,), jnp.float32),
    ] + [pltpu.VMEM((_PER,), jnp.float32)] * 9 + [
        pltpu.VMEM((_NTILES * 16,), jnp.float32),
        pltpu.VMEM((16,), jnp.float32),
        pltpu.VMEM((_MAX_DET * 16,), jnp.float32),
        pltpu.VMEM_SHARED((2 * _NTILES * 16,), jnp.float32),
    ]

    out = pl.kernel(
        functools.partial(_sc_body, n),
        out_type=jax.ShapeDtypeStruct((_MAX_DET * 16,), jnp.float32),
        mesh=plsc.VectorSubcoreMesh(
            core_axis_name="c", subcore_axis_name="s", num_cores=1),
        scratch_types=scratch,
    )(*args)
    return out.reshape(_MAX_DET, 16)[:, :5]
